# Initial kernel scaffold; baseline (speedup 1.0000x reference)
import jax
import jax.numpy as jnp
from jax.experimental import pallas as pl


def kernel(inputs, weight):
    raise NotImplementedError("write your pallas kernel here")


# speedup vs baseline: 6.4678x; 6.4678x over previous
import functools

import jax
import jax.numpy as jnp
from jax import lax
from jax.experimental import pallas as pl
from jax.experimental.pallas import tpu as pltpu
from jax.experimental.pallas import tpu_sc as plsc

N_EMB = 8192
DIM = 256
BETA = 0.25

TM = 256
NCH = 512
BIG = 2**30


def _k1_body(x_ref, x2_ref, wt_ref, w2_ref, enc_ref, idx_ref, cnt_ref,
             msum_ref):
    i = pl.program_id(0)
    x = x_ref[...]
    x2 = x2_ref[...]
    runmin = jnp.full((TM, 1), jnp.inf, dtype=jnp.float32)
    runidx = jnp.zeros((TM, 1), dtype=jnp.int32)
    iota = lax.broadcasted_iota(jnp.int32, (TM, NCH), 1)
    for j in range(N_EMB // NCH):
        wt = wt_ref[:, j * NCH:(j + 1) * NCH]
        w2 = w2_ref[:, j * NCH:(j + 1) * NCH]
        mm = lax.dot_general(x, wt, (((1,), (0,)), ((), ())),
                             preferred_element_type=jnp.float32)
        dist = (x2 + w2) - 2.0 * mm
        cmin = jnp.min(dist, axis=1, keepdims=True)
        cidx = jnp.min(jnp.where(dist == cmin, iota + j * NCH, BIG),
                       axis=1, keepdims=True)
        upd = cmin < runmin
        runidx = jnp.where(upd, cidx, runidx)
        runmin = jnp.where(upd, cmin, runmin)
    idx_ref[...] = runidx

    @pl.when(i == 0)
    def _():
        cnt_ref[...] = jnp.zeros((1, N_EMB), jnp.float32)
        msum_ref[...] = jnp.zeros((1, 1), jnp.float32)

    msum_ref[...] += jnp.sum(runmin, keepdims=True)
    for j in range(N_EMB // NCH):
        oh = (runidx == iota + j * NCH).astype(jnp.float32)
        enc_ref[:, j * NCH:(j + 1) * NCH] = oh
        cnt_ref[:, j * NCH:(j + 1) * NCH] += jnp.sum(oh, axis=0,
                                                     keepdims=True)


def _k1_call(flat, x2, wt, w2):
    nblk = flat.shape[0] // TM
    return pl.pallas_call(
        _k1_body,
        grid=(nblk,),
        in_specs=[
            pl.BlockSpec((TM, DIM), lambda i: (i, 0)),
            pl.BlockSpec((TM, 1), lambda i: (i, 0)),
            pl.BlockSpec((DIM, N_EMB), lambda i: (0, 0)),
            pl.BlockSpec((1, N_EMB), lambda i: (0, 0)),
        ],
        out_specs=[
            pl.BlockSpec((TM, N_EMB), lambda i: (i, 0)),
            pl.BlockSpec((TM, 1), lambda i: (i, 0)),
            pl.BlockSpec((1, N_EMB), lambda i: (0, 0)),
            pl.BlockSpec((1, 1), lambda i: (0, 0)),
        ],
        out_shape=[
            jax.ShapeDtypeStruct((flat.shape[0], N_EMB), jnp.float32),
            jax.ShapeDtypeStruct((flat.shape[0], 1), jnp.int32),
            jax.ShapeDtypeStruct((1, N_EMB), jnp.float32),
            jax.ShapeDtypeStruct((1, 1), jnp.float32),
        ],
        compiler_params=pltpu.CompilerParams(
            dimension_semantics=("arbitrary",)),
    )(flat, x2, wt, w2)


_NC = 2
_NS = 16
_NW = _NC * _NS
_GCH = 128


def _gather_call(weight, idx):
    b = idx.shape[0]
    per_w = b // _NW
    mesh = plsc.VectorSubcoreMesh(core_axis_name="c", subcore_axis_name="s")

    @functools.partial(
        pl.kernel, mesh=mesh,
        out_type=jax.ShapeDtypeStruct((b, DIM), jnp.float32),
        scratch_types=[
            pltpu.VMEM((_GCH,), jnp.int32),
            pltpu.VMEM((_GCH, DIM), jnp.float32),
            pltpu.SemaphoreType.DMA,
        ],
    )
    def _gk(table_hbm, idx_hbm, out_hbm, idx_v, rows_v, sem):
        wid = lax.axis_index("s") * _NC + lax.axis_index("c")
        base = wid * per_w
        for c in range(per_w // _GCH):
            off = base + c * _GCH
            pltpu.sync_copy(idx_hbm.at[pl.ds(off, _GCH)], idx_v)
            pltpu.async_copy(table_hbm.at[idx_v], rows_v, sem).wait()
            pltpu.sync_copy(rows_v, out_hbm.at[pl.ds(off, _GCH)])

    return _gk(weight, idx)


def _k3_body(x_ref, q_ref, o_ref):
    x = x_ref[0]
    q = q_ref[0]
    o_ref[0] = jnp.transpose(x + (q - x), (1, 0))


def _k3_call(x_bt, q_bt):
    b, t, d = x_bt.shape
    return pl.pallas_call(
        _k3_body,
        grid=(b,),
        in_specs=[
            pl.BlockSpec((1, t, d), lambda i: (i, 0, 0)),
            pl.BlockSpec((1, t, d), lambda i: (i, 0, 0)),
        ],
        out_specs=pl.BlockSpec((1, d, t), lambda i: (i, 0, 0)),
        out_shape=jax.ShapeDtypeStruct((b, d, t), jnp.float32),
    )(x_bt, q_bt)


def kernel(inputs, weight):
    b, c, t = inputs.shape
    x = jnp.transpose(inputs, (0, 2, 1))
    flat = x.reshape(-1, DIM)
    n = flat.shape[0]
    x2 = jnp.sum(flat ** 2, axis=1, keepdims=True)
    w2 = jnp.sum(weight ** 2, axis=1)
    wt = weight.T

    enc, idx2d, cnt, msum = _k1_call(
        flat.astype(jnp.bfloat16), x2, wt.astype(jnp.bfloat16),
        w2.reshape(1, N_EMB))
    idx = idx2d.reshape(n)

    quant = _gather_call(weight, idx)
    q_bt = quant.reshape(b, t, DIM)

    quantized_out = _k3_call(x, q_bt)

    m = msum[0, 0] / jnp.float32(n * DIM)
    loss = m + BETA * m
    avg_probs = cnt.reshape(N_EMB) / jnp.float32(n)
    perplexity = jnp.exp(-jnp.sum(avg_probs * jnp.log(avg_probs + 1e-10)))
    return (loss, quantized_out, perplexity, weight, idx.reshape(b, t), enc)
